# baseline (speedup 1.0000x reference)
import functools

import jax
import jax.numpy as jnp
from jax import lax
from jax.experimental import pallas as pl
from jax.experimental.pallas import tpu as pltpu
from jax.experimental.pallas import tpu_sc as plsc

FEAT = 128
TAB = 2 * FEAT
NN = 10000
NE = 320000
EPS = 1e-8
THRESH = 0.1

NC, NS, L = 2, 16, 16
NW = NC * NS
EPW = NE // NW
EC = 80
NCHUNK = EPW // EC


def _tables_body(l_ref, r_ref, w0_ref, w1_ref, a_ref, b_ref):
    w0 = w0_ref[...]
    w1 = w1_ref[...]
    for x_ref, o_ref in ((l_ref, a_ref), (r_ref, b_ref)):
        x = x_ref[...]
        for h, w in enumerate((w0, w1)):
            wx = x * w
            n = jnp.sqrt(jnp.sum(wx * wx, axis=1, keepdims=True))
            o_ref[:, h * FEAT:(h + 1) * FEAT] = wx / jnp.maximum(n, EPS)


def _build_tables(left, right, w0, w1):
    blk = 1000
    grid = NN // blk
    return pl.pallas_call(
        _tables_body,
        grid=(grid,),
        in_specs=[
            pl.BlockSpec((blk, FEAT), lambda i: (i, 0)),
            pl.BlockSpec((blk, FEAT), lambda i: (i, 0)),
            pl.BlockSpec((1, FEAT), lambda i: (0, 0)),
            pl.BlockSpec((1, FEAT), lambda i: (0, 0)),
        ],
        out_specs=[
            pl.BlockSpec((blk, TAB), lambda i: (i, 0)),
            pl.BlockSpec((blk, TAB), lambda i: (i, 0)),
        ],
        out_shape=[
            jax.ShapeDtypeStruct((NN, TAB), jnp.float32),
            jax.ShapeDtypeStruct((NN, TAB), jnp.float32),
        ],
    )(left, right, w0, w1)


def _sc_edges_body(a_hbm, b_hbm, src_hbm, dst_hbm, out_hbm,
                   src_v, dst_v, a_bufs, b_bufs, out_v, sems):
    wid = lax.axis_index("s") * NC + lax.axis_index("c")
    base = wid * EPW
    pltpu.sync_copy(src_hbm.at[pl.ds(base, EPW)], src_v)
    pltpu.sync_copy(dst_hbm.at[pl.ds(base, EPW)], dst_v)

    def issue(c, slot):
        off = c * EC
        pltpu.async_copy(a_hbm.at[src_v.at[pl.ds(off, EC)]], a_bufs[slot],
                         sems[2 * slot])
        pltpu.async_copy(b_hbm.at[dst_v.at[pl.ds(off, EC)]], b_bufs[slot],
                         sems[2 * slot + 1])

    def wait(slot):
        pltpu.make_async_copy(a_hbm.at[pl.ds(0, EC)], a_bufs[slot],
                              sems[2 * slot]).wait()
        pltpu.make_async_copy(b_hbm.at[pl.ds(0, EC)], b_bufs[slot],
                              sems[2 * slot + 1]).wait()

    def compute(c, slot):
        a_buf = a_bufs[slot]
        b_buf = b_bufs[slot]

        def group_body(g, _):
            e0 = g * L
            rows = e0 + lax.iota(jnp.int32, L)
            zero = jnp.full((L,), 0, jnp.int32)
            naccs = 8
            accs = [jnp.zeros((L,), jnp.float32) for _ in range(naccs)]
            for d in range(TAB):
                cold = zero + d
                va = plsc.load_gather(a_buf, [rows, cold])
                vb = plsc.load_gather(b_buf, [rows, cold])
                accs[d % naccs] = accs[d % naccs] + va * vb
            while len(accs) > 1:
                accs = [x + y for x, y in zip(accs[0::2], accs[1::2])]
            s = accs[0] * jnp.float32(0.5)
            res = jnp.where(s < THRESH, jnp.float32(0.0), s)
            out_v[pl.ds(c * EC + e0, L)] = res
            return 0

        lax.fori_loop(0, EC // L, group_body, 0, unroll=False)

    issue(0, 0)

    def pair_body(p, _):
        c0 = 2 * p
        issue(c0 + 1, 1)
        wait(0)
        compute(c0, 0)
        issue(c0 + 2, 0)
        wait(1)
        compute(c0 + 1, 1)
        return 0

    lax.fori_loop(0, (NCHUNK - 1) // 2, pair_body, 0, unroll=False)
    wait(0)
    compute(NCHUNK - 1, 0)

    pltpu.sync_copy(out_v, out_hbm.at[pl.ds(base, EPW)])


@functools.cache
def _sc_edges():
    return pl.kernel(
        _sc_edges_body,
        out_type=jax.ShapeDtypeStruct((NE,), jnp.float32),
        mesh=plsc.VectorSubcoreMesh(core_axis_name="c", subcore_axis_name="s",
                                    num_cores=NC, num_subcores=NS),
        scratch_types=[
            pltpu.VMEM((EPW,), jnp.int32),
            pltpu.VMEM((EPW,), jnp.int32),
            [pltpu.VMEM((EC, TAB), jnp.float32)] * 2,
            [pltpu.VMEM((EC, TAB), jnp.float32)] * 2,
            pltpu.VMEM((EPW,), jnp.float32),
            [pltpu.SemaphoreType.DMA] * 4,
        ],
        compiler_params=pltpu.CompilerParams(use_tc_tiling_on_sc=False,
                                             needs_layout_passes=False,
                                             disable_bounds_checks=True),
    )


def kernel(left_features, right_features, edge_index, W0, W1):
    a_tab, b_tab = _build_tables(left_features, right_features, W0, W1)
    src = edge_index[0]
    dst = edge_index[1]
    return _sc_edges()(a_tab, b_tab, src, dst)


# speedup vs baseline: 6.8374x; 6.8374x over previous
import functools

import jax
import jax.numpy as jnp
from jax import lax
from jax.experimental import pallas as pl
from jax.experimental.pallas import tpu as pltpu
from jax.experimental.pallas import tpu_sc as plsc

FEAT = 128
TAB = 2 * FEAT
NN = 10000
NE = 320000
EPS = 1e-8
THRESH = 0.1

NC, NS, L = 2, 16, 16
NW = NC * NS
EPW = NE // NW
EC = 80
NCHUNK = EPW // EC


def _tables_body(l_ref, r_ref, w0_ref, w1_ref, a_ref, b_ref):
    w0 = w0_ref[...]
    w1 = w1_ref[...]
    for x_ref, o_ref in ((l_ref, a_ref), (r_ref, b_ref)):
        x = x_ref[...]
        for h, w in enumerate((w0, w1)):
            wx = x * w
            n = jnp.sqrt(jnp.sum(wx * wx, axis=1, keepdims=True))
            o_ref[:, h * FEAT:(h + 1) * FEAT] = wx / jnp.maximum(n, EPS)


def _build_tables(left, right, w0, w1):
    blk = 1000
    grid = NN // blk
    return pl.pallas_call(
        _tables_body,
        grid=(grid,),
        in_specs=[
            pl.BlockSpec((blk, FEAT), lambda i: (i, 0)),
            pl.BlockSpec((blk, FEAT), lambda i: (i, 0)),
            pl.BlockSpec((1, FEAT), lambda i: (0, 0)),
            pl.BlockSpec((1, FEAT), lambda i: (0, 0)),
        ],
        out_specs=[
            pl.BlockSpec((blk, TAB), lambda i: (i, 0)),
            pl.BlockSpec((blk, TAB), lambda i: (i, 0)),
        ],
        out_shape=[
            jax.ShapeDtypeStruct((NN, TAB), jnp.float32),
            jax.ShapeDtypeStruct((NN, TAB), jnp.float32),
        ],
    )(left, right, w0, w1)


def _sc_edges_body(a_hbm, b_hbm, src_hbm, dst_hbm, out_hbm,
                   src_v, dst_v, a_bufs, b_bufs, p_buf, out_v, sems):
    wid = lax.axis_index("s") * NC + lax.axis_index("c")
    base = wid * EPW
    pltpu.sync_copy(src_hbm.at[pl.ds(base, EPW)], src_v)
    pltpu.sync_copy(dst_hbm.at[pl.ds(base, EPW)], dst_v)

    def issue(c, slot):
        off = c * EC
        pltpu.async_copy(a_hbm.at[src_v.at[pl.ds(off, EC)]], a_bufs[slot],
                         sems[2 * slot])
        pltpu.async_copy(b_hbm.at[dst_v.at[pl.ds(off, EC)]], b_bufs[slot],
                         sems[2 * slot + 1])

    def wait(slot):
        pltpu.make_async_copy(a_hbm.at[pl.ds(0, EC)], a_bufs[slot],
                              sems[2 * slot]).wait()
        pltpu.make_async_copy(b_hbm.at[pl.ds(0, EC)], b_bufs[slot],
                              sems[2 * slot + 1]).wait()

    def compute(c, slot):
        a_buf = a_bufs[slot]
        b_buf = b_bufs[slot]

        def group_body(g, _):
            e0 = g * L
            rows_t = lax.iota(jnp.int32, L)
            for e in range(L):
                r = e0 + e
                accs = [jnp.zeros((L,), jnp.float32) for _ in range(4)]
                for k in range(TAB // L):
                    va = a_buf[r, pl.ds(k * L, L)]
                    vb = b_buf[r, pl.ds(k * L, L)]
                    accs[k % 4] = accs[k % 4] + va * vb
                acc = (accs[0] + accs[1]) + (accs[2] + accs[3])
                p_buf[e, pl.ds(0, L)] = acc
            sacc = [jnp.zeros((L,), jnp.float32) for _ in range(4)]
            for k in range(L):
                colk = jnp.full((L,), k, jnp.int32)
                sacc[k % 4] = sacc[k % 4] + plsc.load_gather(p_buf, [rows_t, colk])
            s = ((sacc[0] + sacc[1]) + (sacc[2] + sacc[3])) * jnp.float32(0.5)
            res = jnp.where(s < THRESH, jnp.float32(0.0), s)
            out_v[pl.ds(c * EC + e0, L)] = res
            return 0

        lax.fori_loop(0, EC // L, group_body, 0, unroll=False)

    issue(0, 0)

    def pair_body(p, _):
        c0 = 2 * p
        issue(c0 + 1, 1)
        wait(0)
        compute(c0, 0)
        issue(c0 + 2, 0)
        wait(1)
        compute(c0 + 1, 1)
        return 0

    lax.fori_loop(0, (NCHUNK - 1) // 2, pair_body, 0, unroll=False)
    wait(0)
    compute(NCHUNK - 1, 0)

    pltpu.sync_copy(out_v, out_hbm.at[pl.ds(base, EPW)])


@functools.cache
def _sc_edges():
    return pl.kernel(
        _sc_edges_body,
        out_type=jax.ShapeDtypeStruct((NE,), jnp.float32),
        mesh=plsc.VectorSubcoreMesh(core_axis_name="c", subcore_axis_name="s",
                                    num_cores=NC, num_subcores=NS),
        scratch_types=[
            pltpu.VMEM((EPW,), jnp.int32),
            pltpu.VMEM((EPW,), jnp.int32),
            [pltpu.VMEM((EC, TAB), jnp.float32)] * 2,
            [pltpu.VMEM((EC, TAB), jnp.float32)] * 2,
            pltpu.VMEM((L, L + 1), jnp.float32),
            pltpu.VMEM((EPW,), jnp.float32),
            [pltpu.SemaphoreType.DMA] * 4,
        ],
        compiler_params=pltpu.CompilerParams(use_tc_tiling_on_sc=False,
                                             needs_layout_passes=False,
                                             disable_bounds_checks=True),
    )


def kernel(left_features, right_features, edge_index, W0, W1):
    a_tab, b_tab = _build_tables(left_features, right_features, W0, W1)
    src = edge_index[0]
    dst = edge_index[1]
    return _sc_edges()(a_tab, b_tab, src, dst)
